# baseline (speedup 1.0000x reference)
import jax
import jax.numpy as jnp
from jax import lax
from jax.experimental import pallas as pl
from jax.experimental.pallas import tpu as pltpu
from jax.experimental.pallas import tpu_sc as plsc

N_NODES = 10000
N_EDGES = 160000
K = 15
IN_DIM = 128
OUT_DIM = 128
KP_EXTENT = 1.2
EXT_SQ = KP_EXTENT * KP_EXTENT

NC = 2
NS = 16
NW = NC * NS
L = 16

N_PAD = 10240
N_ACC = 10016
DUMMY = 10008
E_PER_TILE = 5056
E_PAD = E_PER_TILE * NW
CHUNK = 16
SUP_CHUNKS = 4
SUP_EDGES = SUP_CHUNKS * CHUNK
N_SUPER = E_PER_TILE // SUP_EDGES
BATCH = 16
TRI_MAX = SUP_EDGES * K + BATCH
ROWS_PER_TILE = N_ACC // NS


def _g_body(f_ref, w_ref, g_ref):
    f = f_ref[...]
    for k in range(K):
        g_ref[:, k * OUT_DIM:(k + 1) * OUT_DIM] = jnp.dot(
            f, w_ref[k], preferred_element_type=jnp.float32
        ).astype(jnp.bfloat16)


def _stage1_g(feats_pad, weights):
    blk = 256
    return pl.pallas_call(
        _g_body,
        grid=(N_PAD // blk,),
        in_specs=[
            pl.BlockSpec((blk, IN_DIM), lambda i: (i, 0)),
            pl.BlockSpec((K, IN_DIM, OUT_DIM), lambda i: (0, 0, 0)),
        ],
        out_specs=pl.BlockSpec((blk, K * OUT_DIM), lambda i: (i, 0)),
        out_shape=jax.ShapeDtypeStruct((N_PAD, K * OUT_DIM), jnp.bfloat16),
    )(feats_pad, weights)


def _dist_from_sq(d2):
    i = plsc.bitcast(d2, jnp.int32)
    i = jnp.int32(0x5F3759DF) - lax.shift_right_logical(i, 1)
    r = plsc.bitcast(i, jnp.float32)
    half = d2 * 0.5
    r = r * (1.5 - half * r * r)
    r = r * (1.5 - half * r * r)
    return d2 * r


def _sc_body(g_hbm, pos_hbm, kp_hbm, edge_hbm, out_hbm,
             px, py, pz, kp_v, pidx, grow, d2buf, dix,
             msgs0, msgs1, rows_a, rows_b, rows_c, rows_d, acc_sh,
             sem_a, sem_b, sem_c, sem_d, sem_s0, sem_s1):
    cid = lax.axis_index("c")
    sid = lax.axis_index("s")
    wid = sid * NC + cid
    base = wid * E_PER_TILE
    row0 = sid * ROWS_PER_TILE

    pltpu.sync_copy(pos_hbm.at[pl.ds(0 * N_PAD, N_ACC)], px)
    pltpu.sync_copy(pos_hbm.at[pl.ds(1 * N_PAD, N_ACC)], py)
    pltpu.sync_copy(pos_hbm.at[pl.ds(2 * N_PAD, N_ACC)], pz)
    pltpu.sync_copy(kp_hbm, kp_v)
    pltpu.sync_copy(edge_hbm.at[pl.ds(base, E_PER_TILE)], pidx)

    zero = jnp.zeros((L,), jnp.float32)

    for r in range(CHUNK):
        for j in range(OUT_DIM // L):
            msgs0[r, pl.ds(j * L, L)] = zero
    for r in range(ROWS_PER_TILE // CHUNK):
        pltpu.async_copy(msgs0, acc_sh.at[pl.ds(row0 + r * CHUNK, CHUNK)],
                         sem_a)
    pltpu.async_copy(msgs0.at[pl.ds(0, ROWS_PER_TILE % CHUNK)],
                     acc_sh.at[pl.ds(row0 + ROWS_PER_TILE - ROWS_PER_TILE % CHUNK,
                                     ROWS_PER_TILE % CHUNK)],
                     sem_a)
    for r in range(ROWS_PER_TILE // CHUNK):
        pltpu.make_async_copy(
            msgs0, acc_sh.at[pl.ds(row0, CHUNK)], sem_a).wait()
    pltpu.make_async_copy(
        msgs0.at[pl.ds(0, ROWS_PER_TILE % CHUNK)],
        acc_sh.at[pl.ds(row0, ROWS_PER_TILE % CHUNK)], sem_a).wait()
    plsc.subcore_barrier()


    lane = lax.iota(jnp.int32, L)
    pad_row = jnp.zeros((L,), jnp.int32)
    pad_d2 = jnp.full((L,), 1.0e6, jnp.float32)
    pad_dst = jnp.full((L,), DUMMY, jnp.int32)

    def issue(b, buf, sem):
        r16 = grow[pl.ds(b * BATCH, BATCH)]
        pltpu.async_copy(g_hbm.at[r16], buf, sem)

    def wait(buf, sem):
        pltpu.make_async_copy(
            g_hbm.at[grow[pl.ds(0, BATCH)]], buf, sem).wait()

    def wait_scatter(ssem):
        pltpu.make_async_copy(msgs0, acc_sh.at[pl.ds(0, CHUNK)], ssem).wait()

    def drain_batch(b, buf, sem, mbuf, ssem):
        wait(buf, sem)

        @pl.when(b >= 2)
        def _():
            wait_scatter(ssem)

        off = b * BATCH
        d2b = d2buf[pl.ds(off, L)]
        hb = jnp.maximum(1.0 - _dist_from_sq(d2b) * (1.0 / KP_EXTENT), 0.0)
        for t in range(L):
            ht = hb[t]
            for grp in range(OUT_DIM // (2 * L)):
                w = buf[t, pl.ds(grp * L, L)]
                a, b2 = plsc.unpack(plsc.bitcast(w, jnp.bfloat16),
                                    format=plsc.PackFormat.INTERLEAVED)
                mbuf[t, pl.ds(grp * 2 * L, L)] = ht * a
                mbuf[t, pl.ds(grp * 2 * L + L, L)] = ht * b2
        d16 = dix[pl.ds(off, L)]
        pltpu.async_copy(mbuf, acc_sh.at[d16], ssem, add=True)

    @pl.loop(0, N_SUPER)
    def super_body(s):
        e0 = s * SUP_EDGES
        ptr = jnp.int32(0)
        for cc in range(SUP_CHUNKS):
            c0 = e0 + cc * CHUNK
            pk = pidx[pl.ds(c0, CHUNK)]
            s16 = lax.shift_right_logical(pk, 14)
            d16 = pk & 16383
            yx = plsc.load_gather(px, [s16]) - plsc.load_gather(px, [d16])
            yy = plsc.load_gather(py, [s16]) - plsc.load_gather(py, [d16])
            yz = plsc.load_gather(pz, [s16]) - plsc.load_gather(pz, [d16])
            rbase = s16 * K
            d2s = []
            counts = jnp.zeros((L,), jnp.int32)
            kv0 = kp_v[pl.ds(0, L)]
            kv1 = kp_v[pl.ds(L, L)]
            kv2 = kp_v[pl.ds(2 * L, L)]
            kvs = (kv0, kv1, kv2)
            for k in range(K):
                dx = yx - kvs[(k * 3 + 0) // L][(k * 3 + 0) % L]
                dy = yy - kvs[(k * 3 + 1) // L][(k * 3 + 1) % L]
                dz = yz - kvs[(k * 3 + 2) // L][(k * 3 + 2) % L]
                d2 = dx * dx + dy * dy + dz * dz
                d2s.append(d2)
                cnt = plsc.all_reduce_population_count(d2 < EXT_SQ)
                counts = jnp.where(lane == k, cnt, counts)
            offs = plsc.cumsum(counts)
            for k in range(K):
                b_k = ptr if k == 0 else ptr + offs[k - 1]
                mask = d2s[k] < EXT_SQ
                plsc.store_compressed(grow.at[pl.ds(b_k, L)], rbase + k,
                                      mask=mask)
                plsc.store_compressed(d2buf.at[pl.ds(b_k, L)], d2s[k],
                                      mask=mask)
                plsc.store_compressed(dix.at[pl.ds(b_k, L)], d16, mask=mask)
            ptr = ptr + offs[K - 1]
        for p in range(BATCH // L):
            grow[pl.ds(ptr + p * L, L)] = pad_row
            d2buf[pl.ds(ptr + p * L, L)] = pad_d2
            dix[pl.ds(ptr + p * L, L)] = pad_dst
        nb = (ptr + (BATCH - 1)) // BATCH
        ring = ((rows_a, sem_a), (rows_b, sem_b),
                (rows_c, sem_c), (rows_d, sem_d))
        for q, (buf, sem) in enumerate(ring):
            @pl.when(q < nb)
            def _(q=q, buf=buf, sem=sem):
                issue(q, buf, sem)

        @pl.loop(0, (nb + 3) // 4)
        def quad_body(p):
            b0 = p * 4
            for q, (buf, sem) in enumerate(ring):
                @pl.when(b0 + q < nb)
                def _(q=q, buf=buf, sem=sem):
                    drain_batch(b0 + q, buf, sem,
                                msgs0 if q % 2 == 0 else msgs1,
                                sem_s0 if q % 2 == 0 else sem_s1)

                    @pl.when(b0 + q + 4 < nb)
                    def _():
                        issue(b0 + q + 4, buf, sem)

        @pl.when(nb > 0)
        def _():
            wait_scatter(sem_s0)

        @pl.when(nb > 1)
        def _():
            wait_scatter(sem_s1)


    plsc.subcore_barrier()
    pltpu.sync_copy(acc_sh.at[pl.ds(row0, ROWS_PER_TILE)],
                    out_hbm.at[cid, pl.ds(row0, ROWS_PER_TILE)])


def _stage2_sc(g_flat, pos_flat, kp_flat, edge_packed):
    mesh = plsc.VectorSubcoreMesh(core_axis_name="c", subcore_axis_name="s")
    kern = pl.kernel(
        _sc_body,
        out_type=jax.ShapeDtypeStruct((NC, N_PAD, OUT_DIM), jnp.float32),
        mesh=mesh,
        scratch_types=[
            pltpu.VMEM((N_ACC,), jnp.float32),
            pltpu.VMEM((N_ACC,), jnp.float32),
            pltpu.VMEM((N_ACC,), jnp.float32),
            pltpu.VMEM((3 * L,), jnp.float32),
            pltpu.VMEM((E_PER_TILE,), jnp.int32),
            pltpu.VMEM((TRI_MAX,), jnp.int32),
            pltpu.VMEM((TRI_MAX,), jnp.float32),
            pltpu.VMEM((TRI_MAX,), jnp.int32),
            pltpu.VMEM((CHUNK, OUT_DIM), jnp.float32),
            pltpu.VMEM((CHUNK, OUT_DIM), jnp.float32),
            pltpu.VMEM((BATCH, OUT_DIM // 2), jnp.float32),
            pltpu.VMEM((BATCH, OUT_DIM // 2), jnp.float32),
            pltpu.VMEM((BATCH, OUT_DIM // 2), jnp.float32),
            pltpu.VMEM((BATCH, OUT_DIM // 2), jnp.float32),
            pltpu.VMEM_SHARED((N_ACC, OUT_DIM), jnp.float32),
            pltpu.SemaphoreType.DMA,
            pltpu.SemaphoreType.DMA,
            pltpu.SemaphoreType.DMA,
            pltpu.SemaphoreType.DMA,
            pltpu.SemaphoreType.DMA,
            pltpu.SemaphoreType.DMA,
        ],
        compiler_params=pltpu.CompilerParams(
            needs_layout_passes=False, use_tc_tiling_on_sc=False),
    )
    return kern(g_flat, pos_flat, kp_flat, edge_packed)


def _add_body(a_ref, b_ref, o_ref):
    o_ref[...] = a_ref[...] + b_ref[...]


def _stage3_add(p0, p1):
    blk = 256
    return pl.pallas_call(
        _add_body,
        grid=(N_PAD // blk,),
        in_specs=[
            pl.BlockSpec((blk, OUT_DIM), lambda i: (i, 0)),
            pl.BlockSpec((blk, OUT_DIM), lambda i: (i, 0)),
        ],
        out_specs=pl.BlockSpec((blk, OUT_DIM), lambda i: (i, 0)),
        out_shape=jax.ShapeDtypeStruct((N_PAD, OUT_DIM), jnp.float32),
    )(p0, p1)


def kernel(feats, pos, edge_index, weights, kernel_points):
    feats = feats.astype(jnp.float32)
    pos = pos.astype(jnp.float32)
    weights = weights.astype(jnp.float32)
    kernel_points = kernel_points.astype(jnp.float32)

    feats_pad = jnp.pad(feats, ((0, N_PAD - N_NODES), (0, 0)))
    order = jnp.asarray(
        [32 * t + 16 * s + j
         for t in range(OUT_DIM // 32) for j in range(16) for s in range(2)],
        dtype=jnp.int32)
    g = _stage1_g(feats_pad, weights[:, :, order])
    g_flat = lax.bitcast_convert_type(
        g.reshape(N_PAD * K, OUT_DIM // 2, 2), jnp.float32)

    pos_flat = jnp.pad(pos, ((0, N_PAD - N_NODES), (0, 0))).T.reshape(-1)
    kp_flat = jnp.pad(kernel_points.reshape(-1), (0, 3 * L - K * 3))

    src = edge_index[0].astype(jnp.int32)
    dst = edge_index[1].astype(jnp.int32)
    src_p = jnp.pad(src, (0, E_PAD - N_EDGES))
    dst_p = jnp.pad(dst, (0, E_PAD - N_EDGES), constant_values=DUMMY)
    edge_packed = (src_p << 14) | dst_p

    partials = _stage2_sc(g_flat, pos_flat, kp_flat, edge_packed)
    out = _stage3_add(partials[0], partials[1])
    return out[:N_NODES]


# speedup vs baseline: 15.1233x; 15.1233x over previous
import jax
import jax.numpy as jnp
from jax import lax
from jax.experimental import pallas as pl
from jax.experimental.pallas import tpu as pltpu
from jax.experimental.pallas import tpu_sc as plsc

N_NODES = 10000
N_EDGES = 160000
K = 15
IN_DIM = 128
OUT_DIM = 128
KP_EXTENT = 1.2
EXT_SQ = KP_EXTENT * KP_EXTENT

NC = 2
NS = 16
NW = NC * NS
L = 16

N_PAD = 10240
N_ACC = 10016
DUMMY = 10008
E_PER_TILE = 5056
E_PAD = E_PER_TILE * NW
CHUNK = 16
SUP_CHUNKS = 4
SUP_EDGES = SUP_CHUNKS * CHUNK
N_SUPER = E_PER_TILE // SUP_EDGES
BATCH = 16
TRI_MAX = SUP_EDGES * K + BATCH
ROWS_PER_TILE = N_ACC // NS


def _g_body(f_ref, wa_ref, wb_ref, g_ref):
    f = f_ref[...]
    for k in range(K):
        ra = jnp.dot(f, wa_ref[k], preferred_element_type=jnp.float32)
        rb = jnp.dot(f, wb_ref[k], preferred_element_type=jnp.float32)
        lo = lax.bitcast_convert_type(
            ra.astype(jnp.bfloat16), jnp.uint16).astype(jnp.uint32)
        hi = lax.bitcast_convert_type(
            rb.astype(jnp.bfloat16), jnp.uint16).astype(jnp.uint32)
        word = lax.bitcast_convert_type(lo | (hi << 16), jnp.float32)
        g_ref[:, k * (OUT_DIM // 2):(k + 1) * (OUT_DIM // 2)] = word


def _stage1_g(feats_pad, w_a, w_b):
    blk = 256
    return pl.pallas_call(
        _g_body,
        grid=(N_PAD // blk,),
        in_specs=[
            pl.BlockSpec((blk, IN_DIM), lambda i: (i, 0)),
            pl.BlockSpec((K, IN_DIM, OUT_DIM // 2), lambda i: (0, 0, 0)),
            pl.BlockSpec((K, IN_DIM, OUT_DIM // 2), lambda i: (0, 0, 0)),
        ],
        out_specs=pl.BlockSpec((blk, K * OUT_DIM // 2), lambda i: (i, 0)),
        out_shape=jax.ShapeDtypeStruct((N_PAD, K * OUT_DIM // 2), jnp.float32),
    )(feats_pad, w_a, w_b)


def _dist_from_sq(d2):
    i = plsc.bitcast(d2, jnp.int32)
    i = jnp.int32(0x5F3759DF) - lax.shift_right_logical(i, 1)
    r = plsc.bitcast(i, jnp.float32)
    half = d2 * 0.5
    r = r * (1.5 - half * r * r)
    r = r * (1.5 - half * r * r)
    return d2 * r


def _sc_body(g_hbm, pos_hbm, kp_hbm, edge_hbm, out_hbm,
             px, py, pz, kp_v, pidx, grow, d2buf, dix,
             msgs0, msgs1, rows_a, rows_b, rows_c, rows_d, acc_sh,
             sem_a, sem_b, sem_c, sem_d, sem_s0, sem_s1):
    cid = lax.axis_index("c")
    sid = lax.axis_index("s")
    wid = sid * NC + cid
    base = wid * E_PER_TILE
    row0 = sid * ROWS_PER_TILE

    pltpu.sync_copy(pos_hbm.at[pl.ds(0 * N_PAD, N_ACC)], px)
    pltpu.sync_copy(pos_hbm.at[pl.ds(1 * N_PAD, N_ACC)], py)
    pltpu.sync_copy(pos_hbm.at[pl.ds(2 * N_PAD, N_ACC)], pz)
    pltpu.sync_copy(kp_hbm, kp_v)
    pltpu.sync_copy(edge_hbm.at[pl.ds(base, E_PER_TILE)], pidx)

    zero = jnp.zeros((L,), jnp.float32)

    for r in range(CHUNK):
        for j in range(OUT_DIM // L):
            msgs0[r, pl.ds(j * L, L)] = zero
    for r in range(ROWS_PER_TILE // CHUNK):
        pltpu.async_copy(msgs0, acc_sh.at[pl.ds(row0 + r * CHUNK, CHUNK)],
                         sem_a)
    pltpu.async_copy(msgs0.at[pl.ds(0, ROWS_PER_TILE % CHUNK)],
                     acc_sh.at[pl.ds(row0 + ROWS_PER_TILE - ROWS_PER_TILE % CHUNK,
                                     ROWS_PER_TILE % CHUNK)],
                     sem_a)
    for r in range(ROWS_PER_TILE // CHUNK):
        pltpu.make_async_copy(
            msgs0, acc_sh.at[pl.ds(row0, CHUNK)], sem_a).wait()
    pltpu.make_async_copy(
        msgs0.at[pl.ds(0, ROWS_PER_TILE % CHUNK)],
        acc_sh.at[pl.ds(row0, ROWS_PER_TILE % CHUNK)], sem_a).wait()
    plsc.subcore_barrier()


    lane = lax.iota(jnp.int32, L)
    pad_row = jnp.zeros((L,), jnp.int32)
    pad_d2 = jnp.full((L,), 1.0e6, jnp.float32)
    pad_dst = jnp.full((L,), DUMMY, jnp.int32)

    def issue(b, buf, sem):
        r16 = grow[pl.ds(b * BATCH, BATCH)]
        pltpu.async_copy(g_hbm.at[r16], buf, sem)

    def wait(buf, sem):
        pltpu.make_async_copy(
            g_hbm.at[grow[pl.ds(0, BATCH)]], buf, sem).wait()

    def wait_scatter(ssem):
        pltpu.make_async_copy(msgs0, acc_sh.at[pl.ds(0, CHUNK)], ssem).wait()

    def drain_batch(b, buf, sem, mbuf, ssem):
        wait(buf, sem)

        @pl.when(b >= 2)
        def _():
            wait_scatter(ssem)

        off = b * BATCH
        d2b = d2buf[pl.ds(off, L)]
        hb = jnp.maximum(1.0 - _dist_from_sq(d2b) * (1.0 / KP_EXTENT), 0.0)
        for t in range(L):
            ht = hb[t]
            for grp in range(OUT_DIM // (2 * L)):
                w = buf[t, pl.ds(grp * L, L)]
                a, b2 = plsc.unpack(plsc.bitcast(w, jnp.bfloat16),
                                    format=plsc.PackFormat.INTERLEAVED)
                mbuf[t, pl.ds(grp * 2 * L, L)] = ht * a
                mbuf[t, pl.ds(grp * 2 * L + L, L)] = ht * b2
        d16 = dix[pl.ds(off, L)]
        pltpu.async_copy(mbuf, acc_sh.at[d16], ssem, add=True)

    @pl.loop(0, N_SUPER)
    def super_body(s):
        e0 = s * SUP_EDGES
        ptr = jnp.int32(0)
        for cc in range(SUP_CHUNKS):
            c0 = e0 + cc * CHUNK
            pk = pidx[pl.ds(c0, CHUNK)]
            s16 = lax.shift_right_logical(pk, 14)
            d16 = pk & 16383
            yx = plsc.load_gather(px, [s16]) - plsc.load_gather(px, [d16])
            yy = plsc.load_gather(py, [s16]) - plsc.load_gather(py, [d16])
            yz = plsc.load_gather(pz, [s16]) - plsc.load_gather(pz, [d16])
            rbase = s16 * K
            d2s = []
            counts = jnp.zeros((L,), jnp.int32)
            kv0 = kp_v[pl.ds(0, L)]
            kv1 = kp_v[pl.ds(L, L)]
            kv2 = kp_v[pl.ds(2 * L, L)]
            kvs = (kv0, kv1, kv2)
            for k in range(K):
                dx = yx - kvs[(k * 3 + 0) // L][(k * 3 + 0) % L]
                dy = yy - kvs[(k * 3 + 1) // L][(k * 3 + 1) % L]
                dz = yz - kvs[(k * 3 + 2) // L][(k * 3 + 2) % L]
                d2 = dx * dx + dy * dy + dz * dz
                d2s.append(d2)
                cnt = plsc.all_reduce_population_count(d2 < EXT_SQ)
                counts = jnp.where(lane == k, cnt, counts)
            offs = plsc.cumsum(counts)
            for k in range(K):
                b_k = ptr if k == 0 else ptr + offs[k - 1]
                mask = d2s[k] < EXT_SQ
                plsc.store_compressed(grow.at[pl.ds(b_k, L)], rbase + k,
                                      mask=mask)
                plsc.store_compressed(d2buf.at[pl.ds(b_k, L)], d2s[k],
                                      mask=mask)
                plsc.store_compressed(dix.at[pl.ds(b_k, L)], d16, mask=mask)
            ptr = ptr + offs[K - 1]
        for p in range(BATCH // L):
            grow[pl.ds(ptr + p * L, L)] = pad_row
            d2buf[pl.ds(ptr + p * L, L)] = pad_d2
            dix[pl.ds(ptr + p * L, L)] = pad_dst
        nb = (ptr + (BATCH - 1)) // BATCH
        ring = ((rows_a, sem_a), (rows_b, sem_b),
                (rows_c, sem_c), (rows_d, sem_d))
        for q, (buf, sem) in enumerate(ring):
            @pl.when(q < nb)
            def _(q=q, buf=buf, sem=sem):
                issue(q, buf, sem)

        @pl.loop(0, (nb + 3) // 4)
        def quad_body(p):
            b0 = p * 4
            for q, (buf, sem) in enumerate(ring):
                @pl.when(b0 + q < nb)
                def _(q=q, buf=buf, sem=sem):
                    drain_batch(b0 + q, buf, sem,
                                msgs0 if q % 2 == 0 else msgs1,
                                sem_s0 if q % 2 == 0 else sem_s1)

                    @pl.when(b0 + q + 4 < nb)
                    def _():
                        issue(b0 + q + 4, buf, sem)

        @pl.when(nb > 0)
        def _():
            wait_scatter(sem_s0)

        @pl.when(nb > 1)
        def _():
            wait_scatter(sem_s1)


    plsc.subcore_barrier()
    pltpu.sync_copy(acc_sh.at[pl.ds(row0, ROWS_PER_TILE)],
                    out_hbm.at[cid, pl.ds(row0, ROWS_PER_TILE)])


def _stage2_sc(g_flat, pos_flat, kp_flat, edge_packed):
    mesh = plsc.VectorSubcoreMesh(core_axis_name="c", subcore_axis_name="s")
    kern = pl.kernel(
        _sc_body,
        out_type=jax.ShapeDtypeStruct((NC, N_PAD, OUT_DIM), jnp.float32),
        mesh=mesh,
        scratch_types=[
            pltpu.VMEM((N_ACC,), jnp.float32),
            pltpu.VMEM((N_ACC,), jnp.float32),
            pltpu.VMEM((N_ACC,), jnp.float32),
            pltpu.VMEM((3 * L,), jnp.float32),
            pltpu.VMEM((E_PER_TILE,), jnp.int32),
            pltpu.VMEM((TRI_MAX,), jnp.int32),
            pltpu.VMEM((TRI_MAX,), jnp.float32),
            pltpu.VMEM((TRI_MAX,), jnp.int32),
            pltpu.VMEM((CHUNK, OUT_DIM), jnp.float32),
            pltpu.VMEM((CHUNK, OUT_DIM), jnp.float32),
            pltpu.VMEM((BATCH, OUT_DIM // 2), jnp.float32),
            pltpu.VMEM((BATCH, OUT_DIM // 2), jnp.float32),
            pltpu.VMEM((BATCH, OUT_DIM // 2), jnp.float32),
            pltpu.VMEM((BATCH, OUT_DIM // 2), jnp.float32),
            pltpu.VMEM_SHARED((N_ACC, OUT_DIM), jnp.float32),
            pltpu.SemaphoreType.DMA,
            pltpu.SemaphoreType.DMA,
            pltpu.SemaphoreType.DMA,
            pltpu.SemaphoreType.DMA,
            pltpu.SemaphoreType.DMA,
            pltpu.SemaphoreType.DMA,
        ],
        compiler_params=pltpu.CompilerParams(
            needs_layout_passes=False, use_tc_tiling_on_sc=False),
    )
    return kern(g_flat, pos_flat, kp_flat, edge_packed)


def _add_body(a_ref, b_ref, o_ref):
    o_ref[...] = a_ref[...] + b_ref[...]


def _stage3_add(p0, p1):
    blk = 256
    return pl.pallas_call(
        _add_body,
        grid=(N_PAD // blk,),
        in_specs=[
            pl.BlockSpec((blk, OUT_DIM), lambda i: (i, 0)),
            pl.BlockSpec((blk, OUT_DIM), lambda i: (i, 0)),
        ],
        out_specs=pl.BlockSpec((blk, OUT_DIM), lambda i: (i, 0)),
        out_shape=jax.ShapeDtypeStruct((N_PAD, OUT_DIM), jnp.float32),
    )(p0, p1)


def kernel(feats, pos, edge_index, weights, kernel_points):
    feats = feats.astype(jnp.float32)
    pos = pos.astype(jnp.float32)
    weights = weights.astype(jnp.float32)
    kernel_points = kernel_points.astype(jnp.float32)

    feats_pad = jnp.pad(feats, ((0, N_PAD - N_NODES), (0, 0)))
    idx_a = jnp.asarray(
        [32 * t + j for t in range(OUT_DIM // 32) for j in range(16)],
        dtype=jnp.int32)
    g = _stage1_g(feats_pad, weights[:, :, idx_a], weights[:, :, idx_a + 16])
    g_flat = g.reshape(N_PAD * K, OUT_DIM // 2)

    pos_flat = jnp.pad(pos, ((0, N_PAD - N_NODES), (0, 0))).T.reshape(-1)
    kp_flat = jnp.pad(kernel_points.reshape(-1), (0, 3 * L - K * 3))

    src = edge_index[0].astype(jnp.int32)
    dst = edge_index[1].astype(jnp.int32)
    src_p = jnp.pad(src, (0, E_PAD - N_EDGES))
    dst_p = jnp.pad(dst, (0, E_PAD - N_EDGES), constant_values=DUMMY)
    edge_packed = (src_p << 14) | dst_p

    partials = _stage2_sc(g_flat, pos_flat, kp_flat, edge_packed)
    out = _stage3_add(partials[0], partials[1])
    return out[:N_NODES]
